# baseline (speedup 1.0000x reference)
import functools
import math

import jax
import jax.numpy as jnp
import numpy as np
from jax import lax
from jax.experimental import pallas as pl
from jax.experimental.pallas import tpu as pltpu
from jax.experimental.pallas import tpu_sc as plsc

VOCAB = 1_000_000
EMBED = 64
MAX_LEN = 512
SEQ = 200
BATCH = 4096

NC = 2
NS = 16
NW = NC * NS

ROWS = SEQ * BATCH
B_PER_W = ROWS // NW
CH = 512
NCH = B_PER_W // CH

TBLK = 512
NTBLK = (VOCAB + 2 * TBLK - 1) // (2 * TBLK)
TROWS = NTBLK * TBLK
HALF = ROWS // 2


def _build_pe_np() -> np.ndarray:
    position = np.arange(0, MAX_LEN, dtype=np.float32)[:, None]
    div_term = np.exp(
        np.arange(0, EMBED, 2, dtype=np.float32) * -(math.log(10000.0) / EMBED)
    )
    pe = np.zeros((MAX_LEN, EMBED), dtype=np.float32)
    pe[:, 0::2] = np.sin(position * div_term)
    pe[:, 1::2] = np.cos(position * div_term)
    return pe[:SEQ]


_PE = _build_pe_np()


def _tc_relayout_body(a_ref, b_ref, o_ref):
    o_ref[:, 0:EMBED] = a_ref[...].T
    o_ref[:, EMBED:128] = b_ref[...].T


_tc_relayout = pl.pallas_call(
    _tc_relayout_body,
    grid=(NTBLK,),
    in_specs=[
        pl.BlockSpec((EMBED, TBLK), lambda g: (0, 2 * g)),
        pl.BlockSpec((EMBED, TBLK), lambda g: (0, 2 * g + 1)),
    ],
    out_specs=pl.BlockSpec((TBLK, 128), lambda g: (g, 0)),
    out_shape=jax.ShapeDtypeStruct((TROWS, 128), jnp.float32),
)


def _sc_body(table_hbm, x_hbm, out_hbm, idx_v, rows_v,
             gsem0, gsem1, wsem0, wsem1):
    gsems = (gsem0, gsem1)
    wsems = (wsem0, wsem1)

    wid = lax.axis_index("s") * NC + lax.axis_index("c")
    base = pl.multiple_of(wid * B_PER_W, B_PER_W)

    pltpu.sync_copy(x_hbm.at[pl.ds(base, B_PER_W)], idx_v)

    def remap(m, carry):
        iv = idx_v[pl.ds(m * 16, 16)]
        hi = lax.bitwise_and(iv, jnp.full((16,), ~1023, jnp.int32))
        lo = lax.shift_left(
            lax.bitwise_and(iv, jnp.full((16,), 511, jnp.int32)),
            jnp.full((16,), 1, jnp.int32))
        h = lax.bitwise_and(
            lax.shift_right_logical(iv, jnp.full((16,), 9, jnp.int32)),
            jnp.full((16,), 1, jnp.int32))
        idx_v[pl.ds(m * 16, 16)] = hi + lo + h
        return carry

    lax.fori_loop(0, B_PER_W // 16, remap, 0, unroll=4)

    def g_copy(c, b):
        start = pl.multiple_of(c * CH, CH)
        return pltpu.make_async_copy(
            table_hbm.at[idx_v.at[pl.ds(start, CH)]], rows_v.at[b], gsems[b]
        )

    def w_copy(c, b):
        r0 = base + c * CH
        half = r0 // HALF
        vrow = pl.multiple_of(r0 - half * HALF, CH)
        return pltpu.make_async_copy(
            rows_v.at[b],
            out_hbm.at[pl.ds(vrow, CH), pl.ds(half * EMBED, EMBED)],
            wsems[b],
        )

    g_copy(0, 0).start()
    g_copy(1, 1).start()

    def step(t, carry):
        for b in range(2):
            c = 2 * t + b
            g_copy(c, b).wait()
            w_copy(c, b).start()

            @pl.when(c + 2 < NCH)
            def _(c=c, b=b):
                w_copy(c, b).wait()
                g_copy(c + 2, b).start()
        return carry

    lax.fori_loop(0, NCH // 2, step, 0)

    w_copy(NCH - 2, 0).wait()
    w_copy(NCH - 1, 1).wait()


@functools.partial(
    pl.kernel,
    out_type=jax.ShapeDtypeStruct((HALF, 128), jnp.float32),
    mesh=plsc.VectorSubcoreMesh(core_axis_name="c", subcore_axis_name="s"),
    compiler_params=pltpu.CompilerParams(use_tc_tiling_on_sc=False),
    scratch_types=[
        pltpu.VMEM((B_PER_W,), jnp.int32),
        pltpu.VMEM((2, CH, EMBED), jnp.float32),
        pltpu.SemaphoreType.DMA,
        pltpu.SemaphoreType.DMA,
        pltpu.SemaphoreType.DMA,
        pltpu.SemaphoreType.DMA,
    ],
)
def _sc_gather(table_hbm, x_hbm, out_hbm, *scratch):
    _sc_body(table_hbm, x_hbm, out_hbm, *scratch)


def _tc_finish_body(v_ref, pe_ref, o_ref):
    s = pl.program_id(0)
    h = s // (SEQ // 2)
    y = v_ref[...].T
    hmask = jnp.broadcast_to(
        (jnp.zeros((EMBED, 1), jnp.int32) + h) == 0, (EMBED, 1024))
    yk = jnp.where(hmask, y[0:EMBED, :], y[EMBED:128, :])
    perow = pe_ref[s, :]
    o_ref[0] = yk + perow[:, None]


_tc_finish = pl.pallas_call(
    _tc_finish_body,
    grid=(SEQ, BATCH // 1024),
    in_specs=[
        pl.BlockSpec((1024, 128),
                     lambda s, b: ((s % (SEQ // 2)) * (BATCH // 1024) + b, 0)),
        pl.BlockSpec((SEQ, EMBED), lambda s, b: (0, 0)),
    ],
    out_specs=pl.BlockSpec((1, EMBED, 1024), lambda s, b: (s, 0, b)),
    out_shape=jax.ShapeDtypeStruct((SEQ, EMBED, BATCH), jnp.float32),
)


@jax.jit
def kernel(x, token_table):
    xf = x.reshape(-1).astype(jnp.int32)
    pe = jnp.asarray(_PE)
    t2 = _tc_relayout(token_table.T, token_table.T)
    tl = t2.reshape(TROWS * 2, EMBED)
    v = _sc_gather(tl, xf)
    out_t = _tc_finish(v, pe)
    return out_t.transpose(0, 2, 1)


# speedup vs baseline: 1.2693x; 1.2693x over previous
import functools
import math

import jax
import jax.numpy as jnp
import numpy as np
from jax import lax
from jax.experimental import pallas as pl
from jax.experimental.pallas import tpu as pltpu
from jax.experimental.pallas import tpu_sc as plsc

VOCAB = 1_000_000
EMBED = 64
MAX_LEN = 512
SEQ = 200
BATCH = 4096

NC = 2
NS = 16
NW = NC * NS

ROWS = SEQ * BATCH
B_PER_W = ROWS // NW
CH = 512
NCH = B_PER_W // CH

TBLK = 512
NTBLK = (VOCAB + 2 * TBLK - 1) // (2 * TBLK)
TROWS = NTBLK * TBLK
HALF = ROWS // 2


def _build_pe_np() -> np.ndarray:
    position = np.arange(0, MAX_LEN, dtype=np.float32)[:, None]
    div_term = np.exp(
        np.arange(0, EMBED, 2, dtype=np.float32) * -(math.log(10000.0) / EMBED)
    )
    pe = np.zeros((MAX_LEN, EMBED), dtype=np.float32)
    pe[:, 0::2] = np.sin(position * div_term)
    pe[:, 1::2] = np.cos(position * div_term)
    return pe[:SEQ]


_PE = _build_pe_np()


def _tc_relayout_body(a_ref, b_ref, o_ref):
    o_ref[:, 0:EMBED] = a_ref[...].T
    o_ref[:, EMBED:128] = b_ref[...].T


_tc_relayout = pl.pallas_call(
    _tc_relayout_body,
    grid=(NTBLK,),
    in_specs=[
        pl.BlockSpec((EMBED, TBLK), lambda g: (0, 2 * g)),
        pl.BlockSpec((EMBED, TBLK), lambda g: (0, 2 * g + 1)),
    ],
    out_specs=pl.BlockSpec((TBLK, 128), lambda g: (g, 0)),
    out_shape=jax.ShapeDtypeStruct((TROWS, 128), jnp.float32),
)


def _sc_body(table_hbm, x_hbm, out_hbm, idx_v, rows_v,
             gsem0, gsem1, wsem0, wsem1):
    gsems = (gsem0, gsem1)
    wsems = (wsem0, wsem1)

    wid = lax.axis_index("s") * NC + lax.axis_index("c")
    base = pl.multiple_of(wid * B_PER_W, B_PER_W)

    pltpu.sync_copy(x_hbm.at[pl.ds(base, B_PER_W)], idx_v)

    def remap(m, carry):
        iv = idx_v[pl.ds(m * 16, 16)]
        hi = lax.bitwise_and(iv, jnp.full((16,), ~(2 * TBLK - 1), jnp.int32))
        lo = lax.shift_left(
            lax.bitwise_and(iv, jnp.full((16,), TBLK - 1, jnp.int32)),
            jnp.full((16,), 1, jnp.int32))
        h = lax.bitwise_and(
            lax.shift_right_logical(iv, jnp.full((16,), 9, jnp.int32)),
            jnp.full((16,), 1, jnp.int32))
        idx_v[pl.ds(m * 16, 16)] = hi + lo + h
        return carry

    lax.fori_loop(0, B_PER_W // 16, remap, 0, unroll=4)

    def g_copy(c, b):
        start = pl.multiple_of(c * CH, CH)
        return pltpu.make_async_copy(
            table_hbm.at[idx_v.at[pl.ds(start, CH)]], rows_v.at[b], gsems[b]
        )

    def w_copy(c, b):
        r0 = base + c * CH
        half = r0 // HALF
        vrow = pl.multiple_of(r0 - half * HALF, CH)
        return pltpu.make_async_copy(
            rows_v.at[b],
            out_hbm.at[pl.ds(vrow, CH), pl.ds(half * EMBED, EMBED)],
            wsems[b],
        )

    g_copy(0, 0).start()
    g_copy(1, 1).start()

    def step(t, carry):
        for b in range(2):
            c = 2 * t + b
            g_copy(c, b).wait()
            w_copy(c, b).start()

            @pl.when(c + 2 < NCH)
            def _(c=c, b=b):
                w_copy(c, b).wait()
                g_copy(c + 2, b).start()
        return carry

    lax.fori_loop(0, NCH // 2, step, 0)

    w_copy(NCH - 2, 0).wait()
    w_copy(NCH - 1, 1).wait()


@functools.partial(
    pl.kernel,
    out_type=jax.ShapeDtypeStruct((HALF, 128), jnp.float32),
    mesh=plsc.VectorSubcoreMesh(core_axis_name="c", subcore_axis_name="s"),
    compiler_params=pltpu.CompilerParams(use_tc_tiling_on_sc=False),
    scratch_types=[
        pltpu.VMEM((B_PER_W,), jnp.int32),
        pltpu.VMEM((2, CH, EMBED), jnp.float32),
        pltpu.SemaphoreType.DMA,
        pltpu.SemaphoreType.DMA,
        pltpu.SemaphoreType.DMA,
        pltpu.SemaphoreType.DMA,
    ],
)
def _sc_gather(table_hbm, x_hbm, out_hbm, *scratch):
    _sc_body(table_hbm, x_hbm, out_hbm, *scratch)


def _tc_finish_body(v_ref, pe_ref, o_ref):
    s = pl.program_id(0)
    h = s // (SEQ // 2)
    y = v_ref[...].T
    hmask = jnp.broadcast_to(
        (jnp.zeros((EMBED, 1), jnp.int32) + h) == 0, (EMBED, BATCH))
    yk = jnp.where(hmask, y[0:EMBED, :], y[EMBED:128, :])
    perow = pe_ref[s, :]
    o_ref[0] = yk + perow[:, None]


_tc_finish = pl.pallas_call(
    _tc_finish_body,
    grid=(SEQ,),
    in_specs=[
        pl.BlockSpec((BATCH, 128), lambda s: (s % (SEQ // 2), 0)),
        pl.BlockSpec((SEQ, EMBED), lambda s: (0, 0)),
    ],
    out_specs=pl.BlockSpec((1, EMBED, BATCH), lambda s: (s, 0, 0)),
    out_shape=jax.ShapeDtypeStruct((SEQ, EMBED, BATCH), jnp.float32),
)


@jax.jit
def kernel(x, token_table):
    xf = x.reshape(-1).astype(jnp.int32)
    pe = jnp.asarray(_PE)
    t2 = _tc_relayout(token_table.T, token_table.T)
    tl = t2.reshape(TROWS * 2, EMBED)
    v = _sc_gather(tl, xf)
    out_t = _tc_finish(v, pe)
    return out_t.transpose(0, 2, 1)


# speedup vs baseline: 1.8665x; 1.4705x over previous
import functools
import math

import jax
import jax.numpy as jnp
import numpy as np
from jax import lax
from jax.experimental import pallas as pl
from jax.experimental.pallas import tpu as pltpu
from jax.experimental.pallas import tpu_sc as plsc

VOCAB = 1_000_000
EMBED = 64
MAX_LEN = 512
SEQ = 200
BATCH = 4096

NC = 2
NS = 16
NW = NC * NS

ROWS = SEQ * BATCH
B_PER_W = ROWS // NW
CH = 512
NCH = B_PER_W // CH

TBLK = 2048
NTBLK = (VOCAB + 2 * TBLK - 1) // (2 * TBLK)
TROWS = NTBLK * TBLK
HALF = ROWS // 2


def _build_pe_np() -> np.ndarray:
    position = np.arange(0, MAX_LEN, dtype=np.float32)[:, None]
    div_term = np.exp(
        np.arange(0, EMBED, 2, dtype=np.float32) * -(math.log(10000.0) / EMBED)
    )
    pe = np.zeros((MAX_LEN, EMBED), dtype=np.float32)
    pe[:, 0::2] = np.sin(position * div_term)
    pe[:, 1::2] = np.cos(position * div_term)
    return pe[:SEQ]


_PE = _build_pe_np()


def _tc_relayout_body(a_ref, b_ref, o_ref):
    o_ref[:, 0:EMBED] = a_ref[...].T
    o_ref[:, EMBED:128] = b_ref[...].T


_tc_relayout = pl.pallas_call(
    _tc_relayout_body,
    grid=(NTBLK,),
    in_specs=[
        pl.BlockSpec((EMBED, TBLK), lambda g: (0, 2 * g)),
        pl.BlockSpec(
            (EMBED, TBLK),
            lambda g: (0, jnp.minimum(2 * g + 1, VOCAB // TBLK))),
    ],
    out_specs=pl.BlockSpec((TBLK, 128), lambda g: (g, 0)),
    out_shape=jax.ShapeDtypeStruct((TROWS, 128), jnp.float32),
)


def _sc_body(table_hbm, x_hbm, out_hbm, idx_v, rows_v,
             gsem0, gsem1, wsem0, wsem1):
    gsems = (gsem0, gsem1)
    wsems = (wsem0, wsem1)

    wid = lax.axis_index("s") * NC + lax.axis_index("c")
    base = pl.multiple_of(wid * B_PER_W, B_PER_W)

    pltpu.sync_copy(x_hbm.at[pl.ds(base, B_PER_W)], idx_v)

    def remap(m, carry):
        iv = idx_v[pl.ds(m * 16, 16)]
        hi = lax.bitwise_and(iv, jnp.full((16,), ~(2 * TBLK - 1), jnp.int32))
        lo = lax.shift_left(
            lax.bitwise_and(iv, jnp.full((16,), TBLK - 1, jnp.int32)),
            jnp.full((16,), 1, jnp.int32))
        h = lax.bitwise_and(
            lax.shift_right_logical(iv, jnp.full((16,), 11, jnp.int32)),
            jnp.full((16,), 1, jnp.int32))
        idx_v[pl.ds(m * 16, 16)] = hi + lo + h
        return carry

    lax.fori_loop(0, B_PER_W // 16, remap, 0, unroll=4)

    def g_copy(c, b):
        start = pl.multiple_of(c * CH, CH)
        return pltpu.make_async_copy(
            table_hbm.at[idx_v.at[pl.ds(start, CH)]], rows_v.at[b], gsems[b]
        )

    def w_copy(c, b):
        r0 = base + c * CH
        half = r0 // HALF
        vrow = pl.multiple_of(r0 - half * HALF, CH)
        return pltpu.make_async_copy(
            rows_v.at[b],
            out_hbm.at[pl.ds(vrow, CH), pl.ds(half * EMBED, EMBED)],
            wsems[b],
        )

    g_copy(0, 0).start()
    g_copy(1, 1).start()

    def step(t, carry):
        for b in range(2):
            c = 2 * t + b
            g_copy(c, b).wait()
            w_copy(c, b).start()

            @pl.when(c + 2 < NCH)
            def _(c=c, b=b):
                w_copy(c, b).wait()
                g_copy(c + 2, b).start()
        return carry

    lax.fori_loop(0, NCH // 2, step, 0)

    w_copy(NCH - 2, 0).wait()
    w_copy(NCH - 1, 1).wait()


@functools.partial(
    pl.kernel,
    out_type=jax.ShapeDtypeStruct((HALF, 128), jnp.float32),
    mesh=plsc.VectorSubcoreMesh(core_axis_name="c", subcore_axis_name="s"),
    compiler_params=pltpu.CompilerParams(use_tc_tiling_on_sc=False),
    scratch_types=[
        pltpu.VMEM((B_PER_W,), jnp.int32),
        pltpu.VMEM((2, CH, EMBED), jnp.float32),
        pltpu.SemaphoreType.DMA,
        pltpu.SemaphoreType.DMA,
        pltpu.SemaphoreType.DMA,
        pltpu.SemaphoreType.DMA,
    ],
)
def _sc_gather(table_hbm, x_hbm, out_hbm, *scratch):
    _sc_body(table_hbm, x_hbm, out_hbm, *scratch)


def _tc_finish_body(v_ref, pe_ref, o_ref):
    s = pl.program_id(0)
    h = s // (SEQ // 2)
    y = v_ref[...].T
    hmask = jnp.broadcast_to(
        (jnp.zeros((EMBED, 1), jnp.int32) + h) == 0, (EMBED, BATCH))
    yk = jnp.where(hmask, y[0:EMBED, :], y[EMBED:128, :])
    perow = pe_ref[s, :]
    o_ref[0] = yk + perow[:, None]


_tc_finish = pl.pallas_call(
    _tc_finish_body,
    grid=(SEQ,),
    in_specs=[
        pl.BlockSpec((BATCH, 128), lambda s: (s % (SEQ // 2), 0)),
        pl.BlockSpec((SEQ, EMBED), lambda s: (0, 0)),
    ],
    out_specs=pl.BlockSpec((1, EMBED, BATCH), lambda s: (s, 0, 0)),
    out_shape=jax.ShapeDtypeStruct((SEQ, EMBED, BATCH), jnp.float32),
)


@jax.jit
def kernel(x, token_table):
    xf = x.reshape(-1).astype(jnp.int32)
    pe = jnp.asarray(_PE)
    t2 = _tc_relayout(token_table.T, token_table.T)
    tl = t2.reshape(TROWS * 2, EMBED)
    v = _sc_gather(tl, xf)
    out_t = _tc_finish(v, pe)
    return out_t.transpose(0, 2, 1)


# speedup vs baseline: 1.9165x; 1.0268x over previous
import functools
import math

import jax
import jax.numpy as jnp
import numpy as np
from jax import lax
from jax.experimental import pallas as pl
from jax.experimental.pallas import tpu as pltpu
from jax.experimental.pallas import tpu_sc as plsc

VOCAB = 1_000_000
EMBED = 64
MAX_LEN = 512
SEQ = 200
BATCH = 4096

NC = 2
NS = 16
NW = NC * NS

ROWS = SEQ * BATCH
B_PER_W = ROWS // NW
CH = 256

TBLK = 2048
NTBLK = (VOCAB + 2 * TBLK - 1) // (2 * TBLK)
TROWS = NTBLK * TBLK
HALF = ROWS // 2


def _build_pe_np() -> np.ndarray:
    position = np.arange(0, MAX_LEN, dtype=np.float32)[:, None]
    div_term = np.exp(
        np.arange(0, EMBED, 2, dtype=np.float32) * -(math.log(10000.0) / EMBED)
    )
    pe = np.zeros((MAX_LEN, EMBED), dtype=np.float32)
    pe[:, 0::2] = np.sin(position * div_term)
    pe[:, 1::2] = np.cos(position * div_term)
    return pe[:SEQ]


_PE = _build_pe_np()


def _tc_relayout_body(a_ref, b_ref, o_ref):
    o_ref[...] = jnp.concatenate([a_ref[...].T, b_ref[...].T], axis=1)


_tc_relayout = pl.pallas_call(
    _tc_relayout_body,
    grid=(NTBLK,),
    in_specs=[
        pl.BlockSpec((EMBED, TBLK), lambda g: (0, 2 * g)),
        pl.BlockSpec(
            (EMBED, TBLK),
            lambda g: (0, jnp.minimum(2 * g + 1, VOCAB // TBLK))),
    ],
    out_specs=pl.BlockSpec((TBLK, 128), lambda g: (g, 0)),
    out_shape=jax.ShapeDtypeStruct((TROWS, 128), jnp.float32),
)


HROWS = ROWS // 2
B_PER_W_H = HROWS // NW
NCH_H = B_PER_W_H // CH
QUART = HROWS // 2


def _sc_body(h, table_hbm, x_hbm, out_hbm, idx_v, rows_v,
             gsem0, gsem1, wsem0, wsem1):
    gsems = (gsem0, gsem1)
    wsems = (wsem0, wsem1)

    wid = lax.axis_index("s") * NC + lax.axis_index("c")
    base = pl.multiple_of(wid * B_PER_W_H, B_PER_W_H)

    pltpu.sync_copy(x_hbm.at[pl.ds(h * HROWS + base, B_PER_W_H)], idx_v)

    def remap(m, carry):
        iv = idx_v[pl.ds(m * 16, 16)]
        hi = lax.bitwise_and(iv, jnp.full((16,), ~(2 * TBLK - 1), jnp.int32))
        lo = lax.shift_left(
            lax.bitwise_and(iv, jnp.full((16,), TBLK - 1, jnp.int32)),
            jnp.full((16,), 1, jnp.int32))
        h = lax.bitwise_and(
            lax.shift_right_logical(iv, jnp.full((16,), 11, jnp.int32)),
            jnp.full((16,), 1, jnp.int32))
        idx_v[pl.ds(m * 16, 16)] = hi + lo + h
        return carry

    lax.fori_loop(0, B_PER_W_H // 16, remap, 0, unroll=4)

    def g_copy(c, b):
        start = pl.multiple_of(c * CH, CH)
        return pltpu.make_async_copy(
            table_hbm.at[idx_v.at[pl.ds(start, CH)]], rows_v.at[b], gsems[b]
        )

    def w_copy(c, b):
        r0 = base + c * CH
        q = r0 // QUART
        vrow = pl.multiple_of(r0 - q * QUART, CH)
        return pltpu.make_async_copy(
            rows_v.at[b],
            out_hbm.at[pl.ds(vrow, CH), pl.ds(q * EMBED, EMBED)],
            wsems[b],
        )

    g_copy(0, 0).start()
    g_copy(1, 1).start()

    def step(t, carry):
        for b in range(2):
            c = 2 * t + b
            g_copy(c, b).wait()
            w_copy(c, b).start()

            @pl.when(c + 2 < NCH_H)
            def _(c=c, b=b):
                w_copy(c, b).wait()
                g_copy(c + 2, b).start()
        return carry

    lax.fori_loop(0, NCH_H // 2, step, 0)

    w_copy(NCH_H - 2, 0).wait()
    w_copy(NCH_H - 1, 1).wait()


def _make_sc_gather(h):
    @functools.partial(
        pl.kernel,
        out_type=jax.ShapeDtypeStruct((QUART, 128), jnp.float32),
        mesh=plsc.VectorSubcoreMesh(core_axis_name="c", subcore_axis_name="s"),
        compiler_params=pltpu.CompilerParams(use_tc_tiling_on_sc=False),
        scratch_types=[
            pltpu.VMEM((B_PER_W_H,), jnp.int32),
            pltpu.VMEM((2, CH, EMBED), jnp.float32),
            pltpu.SemaphoreType.DMA,
            pltpu.SemaphoreType.DMA,
            pltpu.SemaphoreType.DMA,
            pltpu.SemaphoreType.DMA,
        ],
    )
    def _sc_gather(table_hbm, x_hbm, out_hbm, *scratch):
        _sc_body(h, table_hbm, x_hbm, out_hbm, *scratch)

    return _sc_gather


_sc_gather_0 = _make_sc_gather(0)
_sc_gather_1 = _make_sc_gather(1)


def _make_tc_finish(h):
    hs = SEQ // 2

    def _tc_finish_body(v_ref, pe_ref, *rest):
        o_ref = rest[-1]
        sp = pl.program_id(0)
        q = sp // (hs // 2)
        y = v_ref[...].T
        qmask = jnp.broadcast_to(
            (jnp.zeros((EMBED, 1), jnp.int32) + q) == 0, (EMBED, BATCH))
        yk = jnp.where(qmask, y[0:EMBED, :], y[EMBED:128, :])
        perow = pe_ref[sp + h * hs, :]
        o_ref[0] = yk + perow[:, None]

    in_specs = [
        pl.BlockSpec((BATCH, 128), lambda sp: (sp % (hs // 2), 0)),
        pl.BlockSpec((SEQ, EMBED), lambda sp: (0, 0)),
    ]
    kwargs = {}
    if h == 1:
        in_specs.append(pl.BlockSpec(memory_space=pl.ANY))
        kwargs["input_output_aliases"] = {2: 0}

    return pl.pallas_call(
        _tc_finish_body,
        grid=(hs,),
        in_specs=in_specs,
        out_specs=pl.BlockSpec(
            (1, EMBED, BATCH), lambda sp, h=h: (sp + h * hs, 0, 0)),
        out_shape=jax.ShapeDtypeStruct((SEQ, EMBED, BATCH), jnp.float32),
        **kwargs,
    )


_tc_finish_0 = _make_tc_finish(0)
_tc_finish_1 = _make_tc_finish(1)


@jax.jit
def kernel(x, token_table):
    xf = x.reshape(-1).astype(jnp.int32)
    pe = jnp.asarray(_PE)
    t2 = _tc_relayout(token_table.T, token_table.T)
    tl = t2.reshape(TROWS * 2, EMBED)
    v0 = _sc_gather_0(tl, xf)
    v1 = _sc_gather_1(tl, xf)
    o0 = _tc_finish_0(v0, pe)
    out_t = _tc_finish_1(v1, pe, o0)
    return out_t.transpose(0, 2, 1)


# speedup vs baseline: 2.2592x; 1.1788x over previous
import functools
import math

import jax
import jax.numpy as jnp
import numpy as np
from jax import lax
from jax.experimental import pallas as pl
from jax.experimental.pallas import tpu as pltpu
from jax.experimental.pallas import tpu_sc as plsc

VOCAB = 1_000_000
EMBED = 64
MAX_LEN = 512
SEQ = 200
BATCH = 4096

NC = 2
NS = 16
NW = NC * NS

ROWS = SEQ * BATCH
B_PER_W = ROWS // NW
CH = 256

TBLK = 4096
TSHIFT = TBLK.bit_length() - 1
NTBLK = (VOCAB + 2 * TBLK - 1) // (2 * TBLK)
TROWS = NTBLK * TBLK
HALF = ROWS // 2


def _build_pe_np() -> np.ndarray:
    position = np.arange(0, MAX_LEN, dtype=np.float32)[:, None]
    div_term = np.exp(
        np.arange(0, EMBED, 2, dtype=np.float32) * -(math.log(10000.0) / EMBED)
    )
    pe = np.zeros((MAX_LEN, EMBED), dtype=np.float32)
    pe[:, 0::2] = np.sin(position * div_term)
    pe[:, 1::2] = np.cos(position * div_term)
    return pe[:SEQ]


_PE = _build_pe_np()


def _tc_relayout_body(a_ref, b_ref, o_ref):
    o_ref[...] = jnp.concatenate([a_ref[...].T, b_ref[...].T], axis=1)


_tc_relayout = pl.pallas_call(
    _tc_relayout_body,
    grid=(NTBLK,),
    in_specs=[
        pl.BlockSpec((EMBED, TBLK), lambda g: (0, 2 * g)),
        pl.BlockSpec(
            (EMBED, TBLK),
            lambda g: (0, jnp.minimum(2 * g + 1, VOCAB // TBLK))),
    ],
    out_specs=pl.BlockSpec((TBLK, 128), lambda g: (g, 0)),
    out_shape=jax.ShapeDtypeStruct((TROWS, 128), jnp.float32),
)


HROWS = ROWS // 2
B_PER_W_H = HROWS // NW
NCH_H = B_PER_W_H // CH
QUART = HROWS // 2


def _sc_body(h, table_hbm, x_hbm, out_hbm, idx_v, rows_v,
             gsem0, gsem1, wsem0, wsem1):
    gsems = (gsem0, gsem1)
    wsems = (wsem0, wsem1)

    wid = lax.axis_index("s") * NC + lax.axis_index("c")
    base = pl.multiple_of(wid * B_PER_W_H, B_PER_W_H)

    pltpu.sync_copy(x_hbm.at[pl.ds(h * HROWS + base, B_PER_W_H)], idx_v)

    def remap(m, carry):
        iv = idx_v[pl.ds(m * 16, 16)]
        hi = lax.bitwise_and(iv, jnp.full((16,), ~(2 * TBLK - 1), jnp.int32))
        lo = lax.shift_left(
            lax.bitwise_and(iv, jnp.full((16,), TBLK - 1, jnp.int32)),
            jnp.full((16,), 1, jnp.int32))
        h = lax.bitwise_and(
            lax.shift_right_logical(iv, jnp.full((16,), TSHIFT, jnp.int32)),
            jnp.full((16,), 1, jnp.int32))
        idx_v[pl.ds(m * 16, 16)] = hi + lo + h
        return carry

    lax.fori_loop(0, B_PER_W_H // 16, remap, 0, unroll=4)

    def g_copy(c, b):
        start = pl.multiple_of(c * CH, CH)
        return pltpu.make_async_copy(
            table_hbm.at[idx_v.at[pl.ds(start, CH)]], rows_v.at[b], gsems[b]
        )

    def w_copy(c, b):
        r0 = base + c * CH
        q = r0 // QUART
        vrow = pl.multiple_of(r0 - q * QUART, CH)
        return pltpu.make_async_copy(
            rows_v.at[b],
            out_hbm.at[pl.ds(vrow, CH), pl.ds(q * EMBED, EMBED)],
            wsems[b],
        )

    g_copy(0, 0).start()
    g_copy(1, 1).start()

    def step(t, carry):
        for b in range(2):
            c = 2 * t + b
            g_copy(c, b).wait()
            w_copy(c, b).start()

            @pl.when(c + 2 < NCH_H)
            def _(c=c, b=b):
                w_copy(c, b).wait()
                g_copy(c + 2, b).start()
        return carry

    lax.fori_loop(0, NCH_H // 2, step, 0)

    w_copy(NCH_H - 2, 0).wait()
    w_copy(NCH_H - 1, 1).wait()


def _make_sc_gather(h):
    @functools.partial(
        pl.kernel,
        out_type=jax.ShapeDtypeStruct((QUART, 128), jnp.float32),
        mesh=plsc.VectorSubcoreMesh(core_axis_name="c", subcore_axis_name="s"),
        compiler_params=pltpu.CompilerParams(use_tc_tiling_on_sc=False),
        scratch_types=[
            pltpu.VMEM((B_PER_W_H,), jnp.int32),
            pltpu.VMEM((2, CH, EMBED), jnp.float32),
            pltpu.SemaphoreType.DMA,
            pltpu.SemaphoreType.DMA,
            pltpu.SemaphoreType.DMA,
            pltpu.SemaphoreType.DMA,
        ],
    )
    def _sc_gather(table_hbm, x_hbm, out_hbm, *scratch):
        _sc_body(h, table_hbm, x_hbm, out_hbm, *scratch)

    return _sc_gather


_sc_gather_0 = _make_sc_gather(0)
_sc_gather_1 = _make_sc_gather(1)


def _make_tc_finish(h):
    hs = SEQ // 2

    def _tc_finish_body(v_ref, pe_ref, *rest):
        o_ref = rest[-1]
        sp2 = pl.program_id(0)
        q = sp2 // (hs // 4)
        qmask = jnp.broadcast_to(
            (jnp.zeros((EMBED, 1), jnp.int32) + q) == 0, (EMBED, BATCH))
        for u in range(2):
            sp = 2 * sp2 + u
            y = v_ref[pl.ds(u * BATCH, BATCH), :].T
            yk = jnp.where(qmask, y[0:EMBED, :], y[EMBED:128, :])
            perow = pe_ref[sp + h * hs, :]
            o_ref[u] = yk + perow[:, None]

    in_specs = [
        pl.BlockSpec((2 * BATCH, 128), lambda sp2: (sp2 % (hs // 4), 0)),
        pl.BlockSpec((SEQ, EMBED), lambda sp2: (0, 0)),
    ]
    kwargs = {}
    if h == 1:
        in_specs.append(pl.BlockSpec(memory_space=pl.ANY))
        kwargs["input_output_aliases"] = {2: 0}

    return pl.pallas_call(
        _tc_finish_body,
        grid=(hs // 2,),
        in_specs=in_specs,
        out_specs=pl.BlockSpec(
            (2, EMBED, BATCH), lambda sp2, h=h: (sp2 + h * (hs // 2), 0, 0)),
        out_shape=jax.ShapeDtypeStruct((SEQ, EMBED, BATCH), jnp.float32),
        **kwargs,
    )


_tc_finish_0 = _make_tc_finish(0)
_tc_finish_1 = _make_tc_finish(1)


@jax.jit
def kernel(x, token_table):
    xf = x.reshape(-1).astype(jnp.int32)
    pe = jnp.asarray(_PE)
    t2 = _tc_relayout(token_table.T, token_table.T)
    tl = t2.reshape(TROWS * 2, EMBED)
    v0 = _sc_gather_0(tl, xf)
    v1 = _sc_gather_1(tl, xf)
    o0 = _tc_finish_0(v0, pe)
    out_t = _tc_finish_1(v1, pe, o0)
    return out_t.transpose(0, 2, 1)


# speedup vs baseline: 2.3737x; 1.0507x over previous
import functools
import math

import jax
import jax.numpy as jnp
import numpy as np
from jax import lax
from jax.experimental import pallas as pl
from jax.experimental.pallas import tpu as pltpu
from jax.experimental.pallas import tpu_sc as plsc

VOCAB = 1_000_000
EMBED = 64
MAX_LEN = 512
SEQ = 200
BATCH = 4096

NC = 2
NS = 16
NW = NC * NS

ROWS = SEQ * BATCH
B_PER_W = ROWS // NW
CH = 256

TBLK = 8192
TSHIFT = TBLK.bit_length() - 1
NTBLK = (VOCAB + 2 * TBLK - 1) // (2 * TBLK)
TROWS = NTBLK * TBLK
HALF = ROWS // 2


def _build_pe_np() -> np.ndarray:
    position = np.arange(0, MAX_LEN, dtype=np.float32)[:, None]
    div_term = np.exp(
        np.arange(0, EMBED, 2, dtype=np.float32) * -(math.log(10000.0) / EMBED)
    )
    pe = np.zeros((MAX_LEN, EMBED), dtype=np.float32)
    pe[:, 0::2] = np.sin(position * div_term)
    pe[:, 1::2] = np.cos(position * div_term)
    return pe[:SEQ]


_PE = _build_pe_np()


def _tc_relayout_body(a_ref, b_ref, o_ref):
    o_ref[...] = jnp.concatenate([a_ref[...].T, b_ref[...].T], axis=1)


_tc_relayout = pl.pallas_call(
    _tc_relayout_body,
    grid=(NTBLK,),
    in_specs=[
        pl.BlockSpec((EMBED, TBLK), lambda g: (0, 2 * g)),
        pl.BlockSpec(
            (EMBED, TBLK),
            lambda g: (0, jnp.minimum(2 * g + 1, VOCAB // TBLK))),
    ],
    out_specs=pl.BlockSpec((TBLK, 128), lambda g: (g, 0)),
    out_shape=jax.ShapeDtypeStruct((TROWS, 128), jnp.float32),
)


HROWS = ROWS // 2
B_PER_W_H = HROWS // NW
NCH_H = B_PER_W_H // CH
QUART = HROWS // 2


def _sc_body(h, table_hbm, x_hbm, out_hbm, idx_v, rows_v,
             gsem0, gsem1, wsem0, wsem1):
    gsems = (gsem0, gsem1)
    wsems = (wsem0, wsem1)

    wid = lax.axis_index("s") * NC + lax.axis_index("c")
    base = pl.multiple_of(wid * B_PER_W_H, B_PER_W_H)

    pltpu.sync_copy(x_hbm.at[pl.ds(h * HROWS + base, B_PER_W_H)], idx_v)

    def remap(m, carry):
        iv = idx_v[pl.ds(m * 16, 16)]
        hi = lax.bitwise_and(iv, jnp.full((16,), ~(2 * TBLK - 1), jnp.int32))
        lo = lax.shift_left(
            lax.bitwise_and(iv, jnp.full((16,), TBLK - 1, jnp.int32)),
            jnp.full((16,), 1, jnp.int32))
        h = lax.bitwise_and(
            lax.shift_right_logical(iv, jnp.full((16,), TSHIFT, jnp.int32)),
            jnp.full((16,), 1, jnp.int32))
        idx_v[pl.ds(m * 16, 16)] = hi + lo + h
        return carry

    lax.fori_loop(0, B_PER_W_H // 16, remap, 0, unroll=4)

    def g_copy(c, b):
        start = pl.multiple_of(c * CH, CH)
        return pltpu.make_async_copy(
            table_hbm.at[idx_v.at[pl.ds(start, CH)]], rows_v.at[b], gsems[b]
        )

    def w_copy(c, b):
        r0 = base + c * CH
        q = r0 // QUART
        vrow = pl.multiple_of(r0 - q * QUART, CH)
        return pltpu.make_async_copy(
            rows_v.at[b],
            out_hbm.at[pl.ds(vrow, CH), pl.ds(q * EMBED, EMBED)],
            wsems[b],
        )

    g_copy(0, 0).start()
    g_copy(1, 1).start()

    def step(t, carry):
        for b in range(2):
            c = 2 * t + b
            g_copy(c, b).wait()
            w_copy(c, b).start()

            @pl.when(c + 2 < NCH_H)
            def _(c=c, b=b):
                w_copy(c, b).wait()
                g_copy(c + 2, b).start()
        return carry

    lax.fori_loop(0, NCH_H // 2, step, 0)

    w_copy(NCH_H - 2, 0).wait()
    w_copy(NCH_H - 1, 1).wait()


def _make_sc_gather(h):
    @functools.partial(
        pl.kernel,
        out_type=jax.ShapeDtypeStruct((QUART, 128), jnp.float32),
        mesh=plsc.VectorSubcoreMesh(core_axis_name="c", subcore_axis_name="s"),
        compiler_params=pltpu.CompilerParams(use_tc_tiling_on_sc=False),
        scratch_types=[
            pltpu.VMEM((B_PER_W_H,), jnp.int32),
            pltpu.VMEM((2, CH, EMBED), jnp.float32),
            pltpu.SemaphoreType.DMA,
            pltpu.SemaphoreType.DMA,
            pltpu.SemaphoreType.DMA,
            pltpu.SemaphoreType.DMA,
        ],
    )
    def _sc_gather(table_hbm, x_hbm, out_hbm, *scratch):
        _sc_body(h, table_hbm, x_hbm, out_hbm, *scratch)

    return _sc_gather


_sc_gather_0 = _make_sc_gather(0)
_sc_gather_1 = _make_sc_gather(1)


def _make_tc_finish(h):
    hs = SEQ // 2

    def _tc_finish_body(v_ref, pe_ref, *rest):
        o_ref = rest[-1]
        sp2 = pl.program_id(0)
        q = sp2 // (hs // 4)
        qmask = jnp.broadcast_to(
            (jnp.zeros((EMBED, 1), jnp.int32) + q) == 0, (EMBED, BATCH))
        for u in range(2):
            sp = 2 * sp2 + u
            y = v_ref[pl.ds(u * BATCH, BATCH), :].T
            yk = jnp.where(qmask, y[0:EMBED, :], y[EMBED:128, :])
            perow = pe_ref[sp + h * hs, :]
            o_ref[u] = yk + perow[:, None]

    in_specs = [
        pl.BlockSpec((2 * BATCH, 128), lambda sp2: (sp2 % (hs // 4), 0)),
        pl.BlockSpec((SEQ, EMBED), lambda sp2: (0, 0)),
    ]
    kwargs = {}
    if h == 1:
        in_specs.append(pl.BlockSpec(memory_space=pl.ANY))
        kwargs["input_output_aliases"] = {2: 0}

    return pl.pallas_call(
        _tc_finish_body,
        grid=(hs // 2,),
        in_specs=in_specs,
        out_specs=pl.BlockSpec(
            (2, EMBED, BATCH), lambda sp2, h=h: (sp2 + h * (hs // 2), 0, 0)),
        out_shape=jax.ShapeDtypeStruct((SEQ, EMBED, BATCH), jnp.float32),
        **kwargs,
    )


_tc_finish_0 = _make_tc_finish(0)
_tc_finish_1 = _make_tc_finish(1)


@jax.jit
def kernel(x, token_table):
    xf = x.reshape(-1).astype(jnp.int32)
    pe = jnp.asarray(_PE)
    t2 = _tc_relayout(token_table.T, token_table.T)
    tl = t2.reshape(TROWS * 2, EMBED)
    v0 = _sc_gather_0(tl, xf)
    v1 = _sc_gather_1(tl, xf)
    o0 = _tc_finish_0(v0, pe)
    out_t = _tc_finish_1(v1, pe, o0)
    return out_t.transpose(0, 2, 1)


# speedup vs baseline: 2.4335x; 1.0252x over previous
import functools
import math

import jax
import jax.numpy as jnp
import numpy as np
from jax import lax
from jax.experimental import pallas as pl
from jax.experimental.pallas import tpu as pltpu
from jax.experimental.pallas import tpu_sc as plsc

VOCAB = 1_000_000
EMBED = 64
MAX_LEN = 512
SEQ = 200
BATCH = 4096

NC = 2
NS = 16
NW = NC * NS

ROWS = SEQ * BATCH
B_PER_W = ROWS // NW
CH = 256

TBLK = 16384
TSHIFT = TBLK.bit_length() - 1
NTBLK = (VOCAB + 2 * TBLK - 1) // (2 * TBLK)
TROWS = NTBLK * TBLK
HALF = ROWS // 2


def _build_pe_np() -> np.ndarray:
    position = np.arange(0, MAX_LEN, dtype=np.float32)[:, None]
    div_term = np.exp(
        np.arange(0, EMBED, 2, dtype=np.float32) * -(math.log(10000.0) / EMBED)
    )
    pe = np.zeros((MAX_LEN, EMBED), dtype=np.float32)
    pe[:, 0::2] = np.sin(position * div_term)
    pe[:, 1::2] = np.cos(position * div_term)
    return pe[:SEQ]


_PE = _build_pe_np()


def _tc_relayout_body(a_ref, b_ref, o_ref):
    o_ref[...] = jnp.concatenate([a_ref[...].T, b_ref[...].T], axis=1)


_tc_relayout = pl.pallas_call(
    _tc_relayout_body,
    grid=(NTBLK,),
    in_specs=[
        pl.BlockSpec((EMBED, TBLK), lambda g: (0, 2 * g)),
        pl.BlockSpec(
            (EMBED, TBLK),
            lambda g: (0, jnp.minimum(2 * g + 1, VOCAB // TBLK))),
    ],
    out_specs=pl.BlockSpec((TBLK, 128), lambda g: (g, 0)),
    out_shape=jax.ShapeDtypeStruct((TROWS, 128), jnp.float32),
)


HROWS = ROWS // 2
B_PER_W_H = HROWS // NW
NCH_H = B_PER_W_H // CH
QUART = HROWS // 2


def _sc_body(h, table_hbm, x_hbm, out_hbm, idx_v, rows_v,
             gsem0, gsem1, wsem0, wsem1):
    gsems = (gsem0, gsem1)
    wsems = (wsem0, wsem1)

    wid = lax.axis_index("s") * NC + lax.axis_index("c")
    base = pl.multiple_of(wid * B_PER_W_H, B_PER_W_H)

    pltpu.sync_copy(x_hbm.at[pl.ds(h * HROWS + base, B_PER_W_H)], idx_v)

    def remap(m, carry):
        iv = idx_v[pl.ds(m * 16, 16)]
        hi = lax.bitwise_and(iv, jnp.full((16,), ~(2 * TBLK - 1), jnp.int32))
        lo = lax.shift_left(
            lax.bitwise_and(iv, jnp.full((16,), TBLK - 1, jnp.int32)),
            jnp.full((16,), 1, jnp.int32))
        h = lax.bitwise_and(
            lax.shift_right_logical(iv, jnp.full((16,), TSHIFT, jnp.int32)),
            jnp.full((16,), 1, jnp.int32))
        idx_v[pl.ds(m * 16, 16)] = hi + lo + h
        return carry

    lax.fori_loop(0, B_PER_W_H // 16, remap, 0, unroll=4)

    def g_copy(c, b):
        start = pl.multiple_of(c * CH, CH)
        return pltpu.make_async_copy(
            table_hbm.at[idx_v.at[pl.ds(start, CH)]], rows_v.at[b], gsems[b]
        )

    def w_copy(c, b):
        r0 = base + c * CH
        q = r0 // QUART
        vrow = pl.multiple_of(r0 - q * QUART, CH)
        return pltpu.make_async_copy(
            rows_v.at[b],
            out_hbm.at[pl.ds(vrow, CH), pl.ds(q * EMBED, EMBED)],
            wsems[b],
        )

    g_copy(0, 0).start()
    g_copy(1, 1).start()

    def step(t, carry):
        for b in range(2):
            c = 2 * t + b
            g_copy(c, b).wait()
            w_copy(c, b).start()

            @pl.when(c + 2 < NCH_H)
            def _(c=c, b=b):
                w_copy(c, b).wait()
                g_copy(c + 2, b).start()
        return carry

    lax.fori_loop(0, NCH_H // 2, step, 0)

    w_copy(NCH_H - 2, 0).wait()
    w_copy(NCH_H - 1, 1).wait()


def _make_sc_gather(h):
    @functools.partial(
        pl.kernel,
        out_type=jax.ShapeDtypeStruct((QUART, 128), jnp.float32),
        mesh=plsc.VectorSubcoreMesh(core_axis_name="c", subcore_axis_name="s"),
        compiler_params=pltpu.CompilerParams(use_tc_tiling_on_sc=False),
        scratch_types=[
            pltpu.VMEM((B_PER_W_H,), jnp.int32),
            pltpu.VMEM((2, CH, EMBED), jnp.float32),
            pltpu.SemaphoreType.DMA,
            pltpu.SemaphoreType.DMA,
            pltpu.SemaphoreType.DMA,
            pltpu.SemaphoreType.DMA,
        ],
    )
    def _sc_gather(table_hbm, x_hbm, out_hbm, *scratch):
        _sc_body(h, table_hbm, x_hbm, out_hbm, *scratch)

    return _sc_gather


_sc_gather_0 = _make_sc_gather(0)
_sc_gather_1 = _make_sc_gather(1)


def _make_tc_finish(h):
    hs = SEQ // 2

    def _tc_finish_body(v_ref, pe_ref, *rest):
        o_ref = rest[-1]
        sp2 = pl.program_id(0)
        q = sp2 // (hs // 4)
        qmask = jnp.broadcast_to(
            (jnp.zeros((EMBED, 1), jnp.int32) + q) == 0, (EMBED, BATCH))
        for u in range(2):
            sp = 2 * sp2 + u
            y = v_ref[pl.ds(u * BATCH, BATCH), :].T
            yk = jnp.where(qmask, y[0:EMBED, :], y[EMBED:128, :])
            perow = pe_ref[sp + h * hs, :]
            o_ref[u] = yk + perow[:, None]

    in_specs = [
        pl.BlockSpec((2 * BATCH, 128), lambda sp2: (sp2 % (hs // 4), 0)),
        pl.BlockSpec((SEQ, EMBED), lambda sp2: (0, 0)),
    ]
    kwargs = {}
    if h == 1:
        in_specs.append(pl.BlockSpec(memory_space=pl.ANY))
        kwargs["input_output_aliases"] = {2: 0}

    return pl.pallas_call(
        _tc_finish_body,
        grid=(hs // 2,),
        in_specs=in_specs,
        out_specs=pl.BlockSpec(
            (2, EMBED, BATCH), lambda sp2, h=h: (sp2 + h * (hs // 2), 0, 0)),
        out_shape=jax.ShapeDtypeStruct((SEQ, EMBED, BATCH), jnp.float32),
        **kwargs,
    )


_tc_finish_0 = _make_tc_finish(0)
_tc_finish_1 = _make_tc_finish(1)


@jax.jit
def kernel(x, token_table):
    xf = x.reshape(-1).astype(jnp.int32)
    pe = jnp.asarray(_PE)
    t2 = _tc_relayout(token_table.T, token_table.T)
    tl = t2.reshape(TROWS * 2, EMBED)
    v0 = _sc_gather_0(tl, xf)
    v1 = _sc_gather_1(tl, xf)
    o0 = _tc_finish_0(v0, pe)
    out_t = _tc_finish_1(v1, pe, o0)
    return out_t.transpose(0, 2, 1)
